# baseline (speedup 1.0000x reference)
import jax
import jax.numpy as jnp
from jax.experimental import pallas as pl
from jax.experimental.pallas import tpu as pltpu

N, E, A, TK = 16384, 64, 64, 32
R, C = 128, 128


def _body(g_ref, s_ref, wq_ref, wk_ref, wv_ref, bv_ref,
          wp1_ref, bp1_ref, wp2_ref, bp2_ref, wf1_ref, bf1_ref, wf2_ref,
          bf2_ref, out_ref, aw_ref, idx_ref, tw_ref, at_ref):
    g = g_ref[...]
    s = s_ref[...]

    m_mat = jnp.dot(wq_ref[...] * (1.0 / (A ** 0.5)), wk_ref[...].T,
                    preferred_element_type=jnp.float32)
    t = jnp.dot(s, m_mat, preferred_element_type=jnp.float32)
    score = jnp.sum(t * g, axis=1)
    sc2d = score.reshape(R, C)

    mx = jnp.max(sc2d)
    e = jnp.exp(sc2d - mx)
    aw = e / jnp.sum(e)
    aw_ref[...] = aw
    at_ref[...] = aw.T

    BIG = jnp.int32(1 << 30)
    row_i = jax.lax.broadcasted_iota(jnp.int32, (R, C), 0)
    lane = jax.lax.broadcasted_iota(jnp.int32, (1, C), 1)
    lane32 = jax.lax.broadcasted_iota(jnp.int32, (1, TK), 1)

    colmax = jnp.max(aw, axis=0, keepdims=True)
    colrow = jnp.min(jnp.where(aw == colmax, row_i, BIG), axis=0,
                     keepdims=True)

    def step(i, carry):
        cmax, crow, idx_acc, w_acc, ws = carry
        m = jnp.max(cmax)
        fidx = jnp.min(jnp.where(cmax == m, crow * C + lane, BIG))
        c = jnp.bitwise_and(fidx, C - 1)
        r = jnp.right_shift(fidx, 7)
        rowvec = at_ref[pl.ds(c, 1), :]
        rowvec = jnp.where(lane == r, jnp.float32(-1.0), rowvec)
        at_ref[pl.ds(c, 1), :] = rowvec
        nmax = jnp.max(rowvec)
        nrow = jnp.min(jnp.where(rowvec == nmax, lane, BIG))
        sel = lane == c
        cmax = jnp.where(sel, nmax, cmax)
        crow = jnp.where(sel, nrow, crow)
        idx_acc = jnp.where(lane32 == i, fidx, idx_acc)
        w_acc = jnp.where(lane32 == i, m, w_acc)
        ws = ws + m * s_ref[pl.ds(fidx, 1), :]
        return cmax, crow, idx_acc, w_acc, ws

    init = (colmax, colrow, jnp.zeros((1, TK), jnp.int32),
            jnp.zeros((1, TK), jnp.float32), jnp.zeros((1, E), jnp.float32))
    _, _, idx_acc, w_acc, ws = jax.lax.fori_loop(0, TK, step, init)
    idx_ref[...] = idx_acc
    tw_ref[...] = w_acc

    wsum = jnp.sum(w_acc)
    pc0 = jnp.dot(ws, wv_ref[...], preferred_element_type=jnp.float32) \
        + wsum * bv_ref[...]
    h = jnp.maximum(
        jnp.dot(pc0, wp1_ref[...], preferred_element_type=jnp.float32)
        + bp1_ref[...], 0.0)
    pc = jnp.dot(h, wp2_ref[...], preferred_element_type=jnp.float32) \
        + bp2_ref[...]

    c_row = jnp.dot(pc, wf1_ref[E:, :], preferred_element_type=jnp.float32) \
        + bf1_ref[...]
    h2 = jnp.maximum(
        jnp.dot(g, wf1_ref[:E, :], preferred_element_type=jnp.float32)
        + c_row, 0.0)
    out_ref[...] = jnp.dot(h2, wf2_ref[...], preferred_element_type=jnp.float32) \
        + bf2_ref[...]


def kernel(graph_repr, substructure_repr, Wq, bq, Wk, bk, Wv, bv,
           Wp1, bp1, Wp2, bp2, Wf1, bf1, Wf2, bf2):
    out, aw, idx, tw = pl.pallas_call(
        _body,
        out_shape=[
            jax.ShapeDtypeStruct((N, E), jnp.float32),
            jax.ShapeDtypeStruct((R, C), jnp.float32),
            jax.ShapeDtypeStruct((1, TK), jnp.int32),
            jax.ShapeDtypeStruct((1, TK), jnp.float32),
        ],
        scratch_shapes=[pltpu.VMEM((C, R), jnp.float32)],
    )(graph_repr, substructure_repr,
      Wq, Wk, Wv, bv.reshape(1, E),
      Wp1, bp1.reshape(1, A), Wp2, bp2.reshape(1, E),
      Wf1, bf1.reshape(1, A), Wf2, bf2.reshape(1, E))
    return out, aw.reshape(N), idx.reshape(TK), tw.reshape(TK)


# speedup vs baseline: 1.1012x; 1.1012x over previous
import jax
import jax.numpy as jnp
from jax.experimental import pallas as pl

N, E, A, TK = 16384, 64, 64, 32
R, C = 128, 128


def _body(g_ref, s_ref, wq_ref, wk_ref, wv_ref, bv_ref,
          wp1_ref, bp1_ref, wp2_ref, bp2_ref, wf1_ref, bf1_ref, wf2_ref,
          bf2_ref, out_ref, aw_ref, idx_ref, tw_ref):
    g = g_ref[...]
    s = s_ref[...]

    m_mat = jnp.dot(wq_ref[...] * (1.0 / (A ** 0.5)), wk_ref[...].T,
                    preferred_element_type=jnp.float32)
    t = jnp.dot(s, m_mat, preferred_element_type=jnp.float32)
    score = jnp.sum(t * g, axis=1)
    sc2d = score.reshape(R, C)

    mx = jnp.max(sc2d)
    e = jnp.exp(sc2d - mx)
    aw = e / jnp.sum(e)
    aw_ref[...] = aw

    BIG = jnp.int32(1 << 30)
    row_i = jax.lax.broadcasted_iota(jnp.int32, (R, C), 0)
    col_i = jax.lax.broadcasted_iota(jnp.int32, (R, C), 1)
    flat_i = row_i * C + col_i
    lane32 = jax.lax.broadcasted_iota(jnp.int32, (1, TK), 1)

    def step(i, carry):
        a, idx_acc, w_acc = carry
        m = jnp.max(a, keepdims=True)
        mb = jnp.broadcast_to(m, (R, C))
        cand = jnp.where(a == mb, flat_i, BIG)
        fidx = jnp.min(cand, keepdims=True)
        fb = jnp.broadcast_to(fidx, (R, C))
        a = jnp.where(flat_i == fb, jnp.float32(-1.0), a)
        idx_acc = jnp.where(lane32 == i, jnp.broadcast_to(fidx, (1, TK)),
                            idx_acc)
        w_acc = jnp.where(lane32 == i, jnp.broadcast_to(m, (1, TK)), w_acc)
        return a, idx_acc, w_acc

    init = (aw, jnp.zeros((1, TK), jnp.int32),
            jnp.zeros((1, TK), jnp.float32))
    _, idx_acc, w_acc = jax.lax.fori_loop(0, TK, step, init, unroll=True)
    idx_ref[...] = idx_acc
    tw_ref[...] = w_acc

    lane1 = jax.lax.broadcasted_iota(jnp.int32, (1, TK), 1)
    ws = jnp.zeros((1, E), jnp.float32)
    for i in range(TK):
        sel = lane1 == i
        fi = jnp.max(jnp.where(sel, idx_acc, 0))
        wi = jnp.max(jnp.where(sel, w_acc, jnp.float32(0.0)))
        ws = ws + wi * s_ref[pl.ds(fi, 1), :]

    wsum = jnp.sum(w_acc)
    pc0 = jnp.dot(ws, wv_ref[...], preferred_element_type=jnp.float32) \
        + wsum * bv_ref[...]
    h = jnp.maximum(
        jnp.dot(pc0, wp1_ref[...], preferred_element_type=jnp.float32)
        + bp1_ref[...], 0.0)
    pc = jnp.dot(h, wp2_ref[...], preferred_element_type=jnp.float32) \
        + bp2_ref[...]

    c_row = jnp.dot(pc, wf1_ref[E:, :], preferred_element_type=jnp.float32) \
        + bf1_ref[...]
    h2 = jnp.maximum(
        jnp.dot(g, wf1_ref[:E, :], preferred_element_type=jnp.float32)
        + c_row, 0.0)
    out_ref[...] = jnp.dot(h2, wf2_ref[...], preferred_element_type=jnp.float32) \
        + bf2_ref[...]


def kernel(graph_repr, substructure_repr, Wq, bq, Wk, bk, Wv, bv,
           Wp1, bp1, Wp2, bp2, Wf1, bf1, Wf2, bf2):
    out, aw, idx, tw = pl.pallas_call(
        _body,
        out_shape=[
            jax.ShapeDtypeStruct((N, E), jnp.float32),
            jax.ShapeDtypeStruct((R, C), jnp.float32),
            jax.ShapeDtypeStruct((1, TK), jnp.int32),
            jax.ShapeDtypeStruct((1, TK), jnp.float32),
        ],
    )(graph_repr, substructure_repr,
      Wq, Wk, Wv, bv.reshape(1, E),
      Wp1, bp1.reshape(1, A), Wp2, bp2.reshape(1, E),
      Wf1, bf1.reshape(1, A), Wf2, bf2.reshape(1, E))
    return out, aw.reshape(N), idx.reshape(TK), tw.reshape(TK)


# speedup vs baseline: 1.3650x; 1.2395x over previous
import jax
import jax.numpy as jnp
from jax.experimental import pallas as pl
from jax.experimental.pallas import tpu as pltpu

N, E, A, TK = 16384, 64, 64, 32
R, C = 128, 128
NCH = 4
CH = N // NCH
RCH = CH // C


def _body(g_hbm, s_hbm, wq_ref, bq_ref, wk_ref, bk_ref, wv_ref, bv_ref,
          wp1_ref, bp1_ref, wp2_ref, bp2_ref, wf1_ref, bf1_ref, wf2_ref,
          bf2_ref, out_hbm, aw_ref, idx_ref, tw_ref,
          g_v, s_v, o_v, sc_v, *sems):
    g_sems = sems[:NCH]
    s_sems = sems[NCH:2 * NCH]
    o_sems = sems[2 * NCH:]

    g_cps, s_cps = [], []
    for i in range(NCH):
        cp = pltpu.make_async_copy(g_hbm.at[pl.ds(i * CH, CH), :],
                                   g_v.at[pl.ds(i * CH, CH), :], g_sems[i])
        cp.start(); g_cps.append(cp)
        cp = pltpu.make_async_copy(s_hbm.at[pl.ds(i * CH, CH), :],
                                   s_v.at[pl.ds(i * CH, CH), :], s_sems[i])
        cp.start(); s_cps.append(cp)

    for i in range(NCH):
        g_cps[i].wait()
        s_cps[i].wait()
        gi = g_v[pl.ds(i * CH, CH), :]
        si = s_v[pl.ds(i * CH, CH), :]
        q = jnp.dot(si, wq_ref[...], preferred_element_type=jnp.float32) \
            + bq_ref[...]
        k = jnp.dot(gi, wk_ref[...], preferred_element_type=jnp.float32) \
            + bk_ref[...]
        score = jnp.sum(q * k, axis=1) * (1.0 / (A ** 0.5))
        sc_v[pl.ds(i * RCH, RCH), :] = score.reshape(RCH, C)

    sc2d = sc_v[...]
    mx = jnp.max(sc2d)
    ex = jnp.exp(sc2d - mx)
    aw = ex / jnp.sum(ex)
    aw_ref[...] = aw

    BIG = jnp.int32(1 << 30)
    row_i = jax.lax.broadcasted_iota(jnp.int32, (R, C), 0)
    lane = jax.lax.broadcasted_iota(jnp.int32, (1, C), 1)
    lane32 = jax.lax.broadcasted_iota(jnp.int32, (1, TK), 1)

    def step(i, carry):
        a, idx_acc, w_acc = carry
        colmax = jnp.max(a, axis=0, keepdims=True)
        cmb = jnp.broadcast_to(colmax, (R, C))
        colrow = jnp.min(jnp.where(a == cmb, row_i, BIG), axis=0,
                         keepdims=True)
        m = jnp.max(colmax, keepdims=True)
        mb = jnp.broadcast_to(m, (1, C))
        key = jnp.where(colmax == mb, colrow * C + lane, BIG)
        fidx = jnp.min(key, keepdims=True)
        fb = jnp.broadcast_to(fidx, (R, C))
        a = jnp.where(row_i * C + jax.lax.broadcasted_iota(
            jnp.int32, (R, C), 1) == fb, jnp.float32(-1.0), a)
        idx_acc = jnp.where(lane32 == i, jnp.broadcast_to(fidx, (1, TK)),
                            idx_acc)
        w_acc = jnp.where(lane32 == i, jnp.broadcast_to(m, (1, TK)), w_acc)
        return a, idx_acc, w_acc

    init = (aw, jnp.zeros((1, TK), jnp.int32), jnp.zeros((1, TK), jnp.float32))
    _, idx_acc, w_acc = jax.lax.fori_loop(0, TK, step, init)
    idx_ref[...] = idx_acc
    tw_ref[...] = w_acc

    ws = jnp.zeros((1, E), jnp.float32)
    for i in range(TK):
        sel = lane32 == i
        fi = jnp.max(jnp.where(sel, idx_acc, 0))
        wi = jnp.max(jnp.where(sel, w_acc, jnp.float32(0.0)))
        ws = ws + wi * s_v[pl.ds(fi, 1), :]

    wsum = jnp.sum(w_acc)
    pc0 = jnp.dot(ws, wv_ref[...], preferred_element_type=jnp.float32) \
        + wsum * bv_ref[...]
    h = jnp.maximum(
        jnp.dot(pc0, wp1_ref[...], preferred_element_type=jnp.float32)
        + bp1_ref[...], 0.0)
    pc = jnp.dot(h, wp2_ref[...], preferred_element_type=jnp.float32) \
        + bp2_ref[...]
    c_row = jnp.dot(pc, wf1_ref[E:, :], preferred_element_type=jnp.float32) \
        + bf1_ref[...]

    o_cps = []
    for i in range(NCH):
        gi = g_v[pl.ds(i * CH, CH), :]
        h2 = jnp.maximum(
            jnp.dot(gi, wf1_ref[:E, :], preferred_element_type=jnp.float32)
            + c_row, 0.0)
        o_v[pl.ds(i * CH, CH), :] = \
            jnp.dot(h2, wf2_ref[...], preferred_element_type=jnp.float32) \
            + bf2_ref[...]
        cp = pltpu.make_async_copy(o_v.at[pl.ds(i * CH, CH), :],
                                   out_hbm.at[pl.ds(i * CH, CH), :],
                                   o_sems[i])
        cp.start(); o_cps.append(cp)
    for cp in o_cps:
        cp.wait()


def kernel(graph_repr, substructure_repr, Wq, bq, Wk, bk, Wv, bv,
           Wp1, bp1, Wp2, bp2, Wf1, bf1, Wf2, bf2):
    out, aw, idx, tw = pl.pallas_call(
        _body,
        in_specs=[pl.BlockSpec(memory_space=pl.ANY),
                  pl.BlockSpec(memory_space=pl.ANY)]
        + [pl.BlockSpec(x.shape, lambda: (0, 0))
           for x in (Wq, bq.reshape(1, A), Wk, bk.reshape(1, A),
                     Wv, bv.reshape(1, E), Wp1, bp1.reshape(1, A),
                     Wp2, bp2.reshape(1, E), Wf1, bf1.reshape(1, A),
                     Wf2, bf2.reshape(1, E))],
        out_specs=[
            pl.BlockSpec(memory_space=pl.ANY),
            pl.BlockSpec((R, C), lambda: (0, 0)),
            pl.BlockSpec((1, TK), lambda: (0, 0)),
            pl.BlockSpec((1, TK), lambda: (0, 0)),
        ],
        out_shape=[
            jax.ShapeDtypeStruct((N, E), jnp.float32),
            jax.ShapeDtypeStruct((R, C), jnp.float32),
            jax.ShapeDtypeStruct((1, TK), jnp.int32),
            jax.ShapeDtypeStruct((1, TK), jnp.float32),
        ],
        scratch_shapes=[
            pltpu.VMEM((N, E), jnp.float32),
            pltpu.VMEM((N, E), jnp.float32),
            pltpu.VMEM((N, E), jnp.float32),
            pltpu.VMEM((R, C), jnp.float32),
        ] + [pltpu.SemaphoreType.DMA] * (3 * NCH),
    )(graph_repr, substructure_repr,
      Wq, bq.reshape(1, A), Wk, bk.reshape(1, A), Wv, bv.reshape(1, E),
      Wp1, bp1.reshape(1, A), Wp2, bp2.reshape(1, E),
      Wf1, bf1.reshape(1, A), Wf2, bf2.reshape(1, E))
    return out, aw.reshape(N), idx.reshape(TK), tw.reshape(TK)


# speedup vs baseline: 1.3656x; 1.0005x over previous
import jax
import jax.numpy as jnp
from jax.experimental import pallas as pl
from jax.experimental.pallas import tpu as pltpu

N, E, A, TK = 16384, 64, 64, 32
R, C = 128, 128
NCH = 4
CH = N // NCH
RCH = CH // C


def _body(g_hbm, s_hbm, wq_ref, bq_ref, wk_ref, bk_ref, wv_ref, bv_ref,
          wp1_ref, bp1_ref, wp2_ref, bp2_ref, wf1_ref, bf1_ref, wf2_ref,
          bf2_ref, out_hbm, aw_ref, idx_ref, tw_ref,
          g_v, s_v, o_v, sc_v, *sems):
    g_sems = sems[:NCH]
    s_sems = sems[NCH:2 * NCH]
    o_sems = sems[2 * NCH:]

    g_cps, s_cps = [], []
    for i in range(NCH):
        cp = pltpu.make_async_copy(g_hbm.at[pl.ds(i * CH, CH), :],
                                   g_v.at[pl.ds(i * CH, CH), :], g_sems[i])
        cp.start(); g_cps.append(cp)
        cp = pltpu.make_async_copy(s_hbm.at[pl.ds(i * CH, CH), :],
                                   s_v.at[pl.ds(i * CH, CH), :], s_sems[i])
        cp.start(); s_cps.append(cp)

    for i in range(NCH):
        g_cps[i].wait()
        s_cps[i].wait()
        gi = g_v[pl.ds(i * CH, CH), :]
        si = s_v[pl.ds(i * CH, CH), :]
        q = jnp.dot(si, wq_ref[...], preferred_element_type=jnp.float32) \
            + bq_ref[...]
        k = jnp.dot(gi, wk_ref[...], preferred_element_type=jnp.float32) \
            + bk_ref[...]
        score = jnp.sum(q * k, axis=1) * (1.0 / (A ** 0.5))
        sc_v[pl.ds(i * RCH, RCH), :] = score.reshape(RCH, C)

    sc2d = sc_v[...]
    mx = jnp.max(sc2d)
    ex = jnp.exp(sc2d - mx)
    aw = ex / jnp.sum(ex)
    aw_ref[...] = aw

    BIG = jnp.int32(1 << 30)
    row_i = jax.lax.broadcasted_iota(jnp.int32, (R, C), 0)
    col_i = jax.lax.broadcasted_iota(jnp.int32, (R, C), 1)
    flat_i = row_i * C + col_i
    lane = jax.lax.broadcasted_iota(jnp.int32, (1, C), 1)
    lane32 = jax.lax.broadcasted_iota(jnp.int32, (1, TK), 1)

    def step(i, carry):
        a, idx_acc, w_acc = carry
        colmax = jnp.max(a, axis=0, keepdims=True)
        cmb = jnp.broadcast_to(colmax, (R, C))
        colrow = jnp.min(jnp.where(a == cmb, row_i, BIG), axis=0,
                         keepdims=True)
        m = jnp.max(colmax, keepdims=True)
        mb = jnp.broadcast_to(m, (1, C))
        key = jnp.where(colmax == mb, colrow * C + lane, BIG)
        fidx = jnp.min(key, keepdims=True)
        fb = jnp.broadcast_to(fidx, (R, C))
        a = jnp.where(flat_i == fb, jnp.float32(-1.0), a)
        idx_acc = jnp.where(lane32 == i, jnp.broadcast_to(fidx, (1, TK)),
                            idx_acc)
        w_acc = jnp.where(lane32 == i, jnp.broadcast_to(m, (1, TK)), w_acc)
        return a, idx_acc, w_acc

    init = (aw, jnp.zeros((1, TK), jnp.int32), jnp.zeros((1, TK), jnp.float32))
    _, idx_acc, w_acc = jax.lax.fori_loop(0, TK, step, init)
    idx_ref[...] = idx_acc
    tw_ref[...] = w_acc

    ws = jnp.zeros((1, E), jnp.float32)
    for i in range(TK):
        sel = lane32 == i
        fi = jnp.max(jnp.where(sel, idx_acc, 0))
        wi = jnp.max(jnp.where(sel, w_acc, jnp.float32(0.0)))
        ws = ws + wi * s_v[pl.ds(fi, 1), :]

    wsum = jnp.sum(w_acc)
    pc0 = jnp.dot(ws, wv_ref[...], preferred_element_type=jnp.float32) \
        + wsum * bv_ref[...]
    h = jnp.maximum(
        jnp.dot(pc0, wp1_ref[...], preferred_element_type=jnp.float32)
        + bp1_ref[...], 0.0)
    pc = jnp.dot(h, wp2_ref[...], preferred_element_type=jnp.float32) \
        + bp2_ref[...]
    c_row = jnp.dot(pc, wf1_ref[E:, :], preferred_element_type=jnp.float32) \
        + bf1_ref[...]

    o_cps = []
    for i in range(NCH):
        gi = g_v[pl.ds(i * CH, CH), :]
        h2 = jnp.maximum(
            jnp.dot(gi, wf1_ref[:E, :], preferred_element_type=jnp.float32)
            + c_row, 0.0)
        o_v[pl.ds(i * CH, CH), :] = \
            jnp.dot(h2, wf2_ref[...], preferred_element_type=jnp.float32) \
            + bf2_ref[...]
        cp = pltpu.make_async_copy(o_v.at[pl.ds(i * CH, CH), :],
                                   out_hbm.at[pl.ds(i * CH, CH), :],
                                   o_sems[i])
        cp.start(); o_cps.append(cp)
    for cp in o_cps:
        cp.wait()


def kernel(graph_repr, substructure_repr, Wq, bq, Wk, bk, Wv, bv,
           Wp1, bp1, Wp2, bp2, Wf1, bf1, Wf2, bf2):
    out, aw, idx, tw = pl.pallas_call(
        _body,
        in_specs=[pl.BlockSpec(memory_space=pl.ANY),
                  pl.BlockSpec(memory_space=pl.ANY)]
        + [pl.BlockSpec(x.shape, lambda: (0, 0))
           for x in (Wq, bq.reshape(1, A), Wk, bk.reshape(1, A),
                     Wv, bv.reshape(1, E), Wp1, bp1.reshape(1, A),
                     Wp2, bp2.reshape(1, E), Wf1, bf1.reshape(1, A),
                     Wf2, bf2.reshape(1, E))],
        out_specs=[
            pl.BlockSpec(memory_space=pl.ANY),
            pl.BlockSpec((R, C), lambda: (0, 0)),
            pl.BlockSpec((1, TK), lambda: (0, 0)),
            pl.BlockSpec((1, TK), lambda: (0, 0)),
        ],
        out_shape=[
            jax.ShapeDtypeStruct((N, E), jnp.float32),
            jax.ShapeDtypeStruct((R, C), jnp.float32),
            jax.ShapeDtypeStruct((1, TK), jnp.int32),
            jax.ShapeDtypeStruct((1, TK), jnp.float32),
        ],
        scratch_shapes=[
            pltpu.VMEM((N, E), jnp.float32),
            pltpu.VMEM((N, E), jnp.float32),
            pltpu.VMEM((N, E), jnp.float32),
            pltpu.VMEM((R, C), jnp.float32),
        ] + [pltpu.SemaphoreType.DMA] * (3 * NCH),
    )(graph_repr, substructure_repr,
      Wq, bq.reshape(1, A), Wk, bk.reshape(1, A), Wv, bv.reshape(1, E),
      Wp1, bp1.reshape(1, A), Wp2, bp2.reshape(1, E),
      Wf1, bf1.reshape(1, A), Wf2, bf2.reshape(1, E))
    return out, aw.reshape(N), idx.reshape(TK), tw.reshape(TK)
